# baseline (speedup 1.0000x reference)
import functools

import jax
import jax.numpy as jnp
from jax import lax
from jax.experimental import pallas as pl
from jax.experimental.pallas import tpu as pltpu
from jax.experimental.pallas import tpu_sc as plsc

M, D = 16384, 128
N_IN = M * D // 2
N_OUT = M * D

NC, NS, L = 2, 16, 16
NW = NC * NS
CH_IN = N_IN // NW
CH_OUT = N_OUT // NW

SUB = 8
SUB_IN = CH_IN // SUB
SUB_OUT = CH_OUT // SUB
UNROLL = 8

_mesh = plsc.VectorSubcoreMesh(core_axis_name="c", subcore_axis_name="s")


@functools.partial(
    pl.kernel,
    mesh=_mesh,
    out_type=jax.ShapeDtypeStruct((N_OUT,), jnp.float32),
    scratch_types=[
        pltpu.VMEM((CH_IN,), jnp.float32),
        pltpu.VMEM((CH_OUT,), jnp.float32),
        [pltpu.SemaphoreType.DMA] * SUB,
        [pltpu.SemaphoreType.DMA] * SUB,
        [pltpu.SemaphoreType.DMA] * SUB,
    ],
    compiler_params=pltpu.CompilerParams(needs_layout_passes=False),
)
def _extend_sc(x_hbm, nan_hbm, out_hbm, in_v, out_v, in_sems, nan_sems, out_sems):
    wid = lax.axis_index("s") * NC + lax.axis_index("c")
    base_in = wid * CH_IN
    base_out = wid * CH_OUT

    odd = lax.iota(jnp.int32, L) * 2 + 1

    nan_copies = []
    in_copies = []
    for s in range(SUB):
        nan_copies.append(
            pltpu.async_copy(
                nan_hbm, out_v.at[pl.ds(s * SUB_OUT, SUB_OUT)], nan_sems[s]
            )
        )
        in_copies.append(
            pltpu.async_copy(
                x_hbm.at[pl.ds(base_in + s * SUB_IN, SUB_IN)],
                in_v.at[pl.ds(s * SUB_IN, SUB_IN)],
                in_sems[s],
            )
        )

    def make_body(sub_base):
        def body(i, carry):
            b = sub_base + i * (L * UNROLL)
            for u in range(UNROLL):
                w = in_v[pl.ds(b + u * L, L)]
                o = 2 * b + u * (2 * L)
                plsc.store_scatter(out_v, [o + odd], w)
            return carry

        return body

    out_copies = []
    for s in range(SUB):
        nan_copies[s].wait()
        in_copies[s].wait()
        lax.fori_loop(0, SUB_IN // (L * UNROLL), make_body(s * SUB_IN), 0)
        out_copies.append(
            pltpu.async_copy(
                out_v.at[pl.ds(s * SUB_OUT, SUB_OUT)],
                out_hbm.at[pl.ds(base_out + s * SUB_OUT, SUB_OUT)],
                out_sems[s],
            )
        )
    for c in out_copies:
        c.wait()


def kernel(x):
    nan_src = jnp.full((SUB_OUT,), jnp.nan, dtype=jnp.float32)
    out_flat = _extend_sc(x.reshape(-1), nan_src)
    return out_flat.reshape(M, D)


# speedup vs baseline: 1.4753x; 1.4753x over previous
import functools

import jax
import jax.numpy as jnp
from jax import lax
from jax.experimental import pallas as pl
from jax.experimental.pallas import tpu as pltpu
from jax.experimental.pallas import tpu_sc as plsc

M, D = 16384, 128
N_IN = M * D // 2
N_OUT = M * D

NC, NS, L = 2, 16, 16
NW = NC * NS
CH_IN = N_IN // NW
CH_OUT = N_OUT // NW

SUB = 8
SUB_IN = CH_IN // SUB
SUB_OUT = CH_OUT // SUB
UNROLL = 8

_mesh = plsc.VectorSubcoreMesh(core_axis_name="c", subcore_axis_name="s")


@functools.partial(
    pl.kernel,
    mesh=_mesh,
    out_type=jax.ShapeDtypeStruct((N_OUT,), jnp.float32),
    scratch_types=[
        pltpu.VMEM((CH_IN,), jnp.float32),
        pltpu.VMEM((CH_OUT,), jnp.float32),
        [pltpu.SemaphoreType.DMA] * SUB,
        [pltpu.SemaphoreType.DMA] * SUB,
    ],
    compiler_params=pltpu.CompilerParams(needs_layout_passes=False),
)
def _extend_sc(x_hbm, out_hbm, in_v, out_v, in_sems, out_sems):
    wid = lax.axis_index("s") * NC + lax.axis_index("c")
    base_in = wid * CH_IN
    base_out = wid * CH_OUT

    nan_vec = jnp.full((L,), jnp.nan, dtype=jnp.float32)
    odd = lax.iota(jnp.int32, L) * 2 + 1

    in_copies = [
        pltpu.async_copy(
            x_hbm.at[pl.ds(base_in + s * SUB_IN, SUB_IN)],
            in_v.at[pl.ds(s * SUB_IN, SUB_IN)],
            in_sems[s],
        )
        for s in range(SUB)
    ]

    def make_body(sub_base):
        def body(i, carry):
            b = sub_base + i * (L * UNROLL)
            for u in range(UNROLL):
                w = in_v[pl.ds(b + u * L, L)]
                o = 2 * b + u * (2 * L)
                out_v[pl.ds(o, L)] = nan_vec
                out_v[pl.ds(o + L, L)] = nan_vec
                plsc.store_scatter(out_v, [o + odd], w)
            return carry

        return body

    out_copies = []
    for s in range(SUB):
        in_copies[s].wait()
        lax.fori_loop(0, SUB_IN // (L * UNROLL), make_body(s * SUB_IN), 0)
        out_copies.append(
            pltpu.async_copy(
                out_v.at[pl.ds(s * SUB_OUT, SUB_OUT)],
                out_hbm.at[pl.ds(base_out + s * SUB_OUT, SUB_OUT)],
                out_sems[s],
            )
        )
    for c in out_copies:
        c.wait()


def kernel(x):
    out_flat = _extend_sc(x.reshape(-1))
    return out_flat.reshape(M, D)
